# baseline (speedup 1.0000x reference)
import functools

import jax
import jax.numpy as jnp
from jax import lax
from jax.experimental import pallas as pl
from jax.experimental.pallas import tpu as pltpu
from jax.experimental.pallas import tpu_sc as plsc

NC = 2
NS = 16
L = 16
NW = NC * NS
D = 64
CHUNK = 128
MARGIN = 24.0


def _sc_distances(ent, rel, hp, rp, tp, hn, rn, tn, w):
    B = hp.shape[0]
    NNEG = hn.shape[0]
    pos_per_w = B // NW
    neg_per_w = NNEG // NW
    n_chunks = neg_per_w // CHUNK

    mesh = plsc.VectorSubcoreMesh(
        core_axis_name="c", subcore_axis_name="s",
        num_cores=NC, num_subcores=NS)

    @functools.partial(
        pl.kernel,
        out_type=(
            jax.ShapeDtypeStruct((B,), jnp.float32),
            jax.ShapeDtypeStruct((B,), jnp.float32),
            jax.ShapeDtypeStruct((NNEG,), jnp.float32),
        ),
        mesh=mesh,
        scratch_types=[
            pltpu.VMEM((CHUNK,), jnp.int32),
            pltpu.VMEM((CHUNK,), jnp.int32),
            pltpu.VMEM((CHUNK,), jnp.int32),
            pltpu.VMEM((CHUNK, D), jnp.float32),
            pltpu.VMEM((CHUNK, D), jnp.float32),
            pltpu.VMEM((CHUNK, D), jnp.float32),
            pltpu.VMEM((2048,), jnp.float32),
            pltpu.VMEM((128,), jnp.float32),
            pltpu.VMEM((D,), jnp.float32),
            pltpu.SemaphoreType.DMA,
            pltpu.SemaphoreType.DMA,
            pltpu.SemaphoreType.DMA,
        ],
        compiler_params=pltpu.CompilerParams(use_tc_tiling_on_sc=False),
    )
    def k(ent_h, rel_h, hp_h, rp_h, tp_h, hn_h, rn_h, tn_h, w_h,
          pos_dist_h, pos_dot_h, neg_dist_h,
          idxh, idxr, idxt, hb, rb, tb, db, dotb, wv,
          sem1, sem2, sem3):
        wid = lax.axis_index("s") * NC + lax.axis_index("c")
        lanes = lax.iota(jnp.int32, L)
        rots = [(lanes + sh) % L for sh in (8, 4, 2, 1)]

        def hsum(v):
            for idc in rots:
                v = v + v.at[idc].get(mode="promise_in_bounds")
            return v

        def gather_chunk(src_h, src_r, src_t, base):
            pltpu.sync_copy(src_h.at[pl.ds(base, CHUNK)], idxh)
            pltpu.sync_copy(src_r.at[pl.ds(base, CHUNK)], idxr)
            pltpu.sync_copy(src_t.at[pl.ds(base, CHUNK)], idxt)
            c1 = pltpu.async_copy(ent_h.at[idxh], hb, sem1)
            c2 = pltpu.async_copy(rel_h.at[idxr], rb, sem2)
            c3 = pltpu.async_copy(ent_h.at[idxt], tb, sem3)
            c1.wait()
            c2.wait()
            c3.wait()

        def abs_sums(i):
            acc = None
            for k4 in range(4):
                sl = pl.ds(k4 * L, L)
                s = hb[i, sl] + rb[i, sl] - tb[i, sl]
                a = jnp.abs(s)
                acc = a if acc is None else acc + a
            return acc

        def neg_chunk(c, _):
            base = wid * neg_per_w + c * CHUNK
            gather_chunk(hn_h, rn_h, tn_h, base)

            def group(g, _):
                i0 = g * L
                dvec = jnp.zeros((L,), jnp.float32)
                for u in range(L):
                    tot = hsum(abs_sums(i0 + u))
                    dvec = jnp.where(lanes == u, tot, dvec)
                db[pl.ds(c * CHUNK + i0, L)] = dvec
                return 0

            lax.fori_loop(0, CHUNK // L, group, 0)
            return 0

        lax.fori_loop(0, n_chunks, neg_chunk, 0)
        pltpu.sync_copy(db, neg_dist_h.at[pl.ds(wid * neg_per_w, neg_per_w)])

        pltpu.sync_copy(w_h, wv)
        pbase = wid * pos_per_w
        gather_chunk(hp_h, rp_h, tp_h, pbase)
        wregs = [wv[pl.ds(k4 * L, L)] for k4 in range(4)]

        def pgroup(g, _):
            i0 = g * L
            dvec = jnp.zeros((L,), jnp.float32)
            mvec = jnp.zeros((L,), jnp.float32)
            for u in range(L):
                i = i0 + u
                dacc = None
                macc = None
                for k4 in range(4):
                    sl = pl.ds(k4 * L, L)
                    s = hb[i, sl] + rb[i, sl] - tb[i, sl]
                    a = jnp.abs(s)
                    m = s * wregs[k4]
                    dacc = a if dacc is None else dacc + a
                    macc = m if macc is None else macc + m
                dvec = jnp.where(lanes == u, hsum(dacc), dvec)
                mvec = jnp.where(lanes == u, hsum(macc), mvec)
            db[pl.ds(i0, L)] = dvec
            dotb[pl.ds(i0, L)] = mvec
            return 0

        lax.fori_loop(0, pos_per_w // L, pgroup, 0)
        pltpu.sync_copy(db.at[pl.ds(0, pos_per_w)],
                        pos_dist_h.at[pl.ds(pbase, pos_per_w)])
        pltpu.sync_copy(dotb, pos_dot_h.at[pl.ds(pbase, pos_per_w)])

    return k(ent, rel, hp, rp, tp, hn, rn, tn, w)


def kernel(positive_triples, block_of_negative_triples, negative_sample_size,
           entities_emb, relations_emb, D_W, D_b):
    B = positive_triples.shape[0]
    neg = block_of_negative_triples.shape[1]
    hp = positive_triples[:, 0]
    rp = positive_triples[:, 1]
    tp = positive_triples[:, 2]
    negf = block_of_negative_triples.reshape(B * neg, 3)
    hn = negf[:, 0]
    rn = negf[:, 1]
    tn = negf[:, 2]
    w = D_W.reshape(-1)

    pos_dist, pos_dot, neg_dist = _sc_distances(
        entities_emb, relations_emb, hp, rp, tp, hn, rn, tn, w)

    confident_scores = jax.nn.sigmoid(pos_dot + D_b[0])
    pos_scores = -jnp.log(jax.nn.sigmoid(MARGIN - pos_dist))
    nd = neg_dist.reshape(B, neg)
    neg_scores = jnp.sum(
        (1.0 / negative_sample_size) * jnp.log(jax.nn.sigmoid(MARGIN - nd)),
        axis=1)
    return confident_scores * (pos_scores + neg_scores)
